# Initial kernel scaffold; baseline (speedup 1.0000x reference)
import jax
import jax.numpy as jnp
from jax.experimental import pallas as pl


def kernel(data):
    raise NotImplementedError("write your pallas kernel here")


# speedup vs baseline: 16.6233x; 16.6233x over previous
import jax
import jax.numpy as jnp
from jax.experimental import pallas as pl

_STD = 3.0
_MAX_SHIFT = 6


def _tshift_body(s_ref, x_ref, o_ref):
    x = x_ref[0]
    a = s_ref[0] + _MAX_SHIFT
    y = jnp.roll(x, -_MAX_SHIFT, axis=0)
    for k in (1, 2, 4, 8):
        m = (a & k) != 0
        y = jnp.where(m, jnp.roll(y, k, axis=0), y)
    o_ref[0] = y


def kernel(data):
    B, T, C = data.shape
    skey = jax.random.key(42)
    shifts = jax.random.normal(skey, (B, 1, C), dtype=jnp.float32) * _STD
    shifts = jnp.clip(jnp.round(shifts).astype(jnp.int32), -_MAX_SHIFT, _MAX_SHIFT)
    shifts = shifts.reshape(B, 1, C)
    return pl.pallas_call(
        _tshift_body,
        grid=(B,),
        in_specs=[
            pl.BlockSpec((1, 1, C), lambda b: (b, 0, 0)),
            pl.BlockSpec((1, T, C), lambda b: (b, 0, 0)),
        ],
        out_specs=pl.BlockSpec((1, T, C), lambda b: (b, 0, 0)),
        out_shape=jax.ShapeDtypeStruct((B, T, C), data.dtype),
    )(shifts, data)
